# Initial kernel scaffold; baseline (speedup 1.0000x reference)
import jax
import jax.numpy as jnp
from jax.experimental import pallas as pl


def kernel(x, adj, W1, b1, W2, b2):
    raise NotImplementedError("write your pallas kernel here")


# speedup vs baseline: 46.3408x; 46.3408x over previous
import jax
import jax.numpy as jnp
from jax.experimental import pallas as pl
from jax.experimental.pallas import tpu as pltpu


def _row_tile(n):
    for t in (200, 128, 80, 64, 40, 32, 16, 8):
        if n % t == 0:
            return t
    return n


def _colsum_body(adj_ref, dinv_ref):
    i = pl.program_id(0)

    @pl.when(i == 0)
    def _init():
        dinv_ref[...] = jnp.zeros_like(dinv_ref)

    dinv_ref[...] += jnp.sum(adj_ref[...], axis=0, keepdims=True)

    @pl.when(i == pl.num_programs(0) - 1)
    def _fin():
        dinv_ref[...] = jax.lax.rsqrt(dinv_ref[...] + 1.0)


def _agg1_body(adj_ref, x_ref, w1_ref, b1_ref, w2_ref, dinv_ref,
               dinv_full_ref, out_ref, acc_ref):
    i = pl.program_id(0)
    tr = adj_ref.shape[0]

    @pl.when(i == 0)
    def _init():
        acc_ref[...] = jnp.zeros_like(acc_ref)

    u = jnp.dot(x_ref[...], w1_ref[...],
                preferred_element_type=jnp.float32) * dinv_ref[...]
    acc_ref[...] += jax.lax.dot_general(
        adj_ref[...], u, (((0,), (0,)), ((), ())),
        preferred_element_type=jnp.float32)
    acc_ref[pl.ds(i * tr, tr), :] += u

    @pl.when(i == pl.num_programs(0) - 1)
    def _fin():
        o1 = jnp.maximum(acc_ref[...] * dinv_full_ref[...] + b1_ref[...], 0.0)
        out_ref[...] = jnp.dot(o1, w2_ref[...],
                               preferred_element_type=jnp.float32) * dinv_full_ref[...]


def _agg2_body(adj_ref, u2_ref, b2_ref, dinv_full_ref, out_ref, acc_ref):
    i = pl.program_id(0)
    tr = adj_ref.shape[0]

    @pl.when(i == 0)
    def _init():
        acc_ref[...] = jnp.zeros_like(acc_ref)

    u = u2_ref[...]
    acc_ref[...] += jax.lax.dot_general(
        adj_ref[...], u, (((0,), (0,)), ((), ())),
        preferred_element_type=jnp.float32)
    acc_ref[pl.ds(i * tr, tr), :] += u

    @pl.when(i == pl.num_programs(0) - 1)
    def _fin():
        logits = acc_ref[...] * dinv_full_ref[...] + b2_ref[...]
        m = jnp.max(logits, axis=1, keepdims=True)
        s = logits - m
        lse = jnp.log(jnp.sum(jnp.exp(s), axis=1, keepdims=True))
        out_ref[...] = s - lse


def kernel(x, adj, W1, b1, W2, b2):
    n, d = x.shape
    h = W1.shape[1]
    c = W2.shape[1]
    tr = _row_tile(n)
    grid = (n // tr,)

    dinv = pl.pallas_call(
        _colsum_body,
        grid=grid,
        in_specs=[pl.BlockSpec((tr, n), lambda i: (i, 0))],
        out_specs=pl.BlockSpec((1, n), lambda i: (0, 0)),
        out_shape=jax.ShapeDtypeStruct((1, n), jnp.float32),
    )(adj)
    dinv_col = dinv.reshape(n, 1)

    u2 = pl.pallas_call(
        _agg1_body,
        grid=grid,
        in_specs=[
            pl.BlockSpec((tr, n), lambda i: (i, 0)),
            pl.BlockSpec((tr, d), lambda i: (i, 0)),
            pl.BlockSpec((d, h), lambda i: (0, 0)),
            pl.BlockSpec((1, h), lambda i: (0, 0)),
            pl.BlockSpec((h, c), lambda i: (0, 0)),
            pl.BlockSpec((tr, 1), lambda i: (i, 0)),
            pl.BlockSpec((n, 1), lambda i: (0, 0)),
        ],
        out_specs=pl.BlockSpec((n, c), lambda i: (0, 0)),
        out_shape=jax.ShapeDtypeStruct((n, c), jnp.float32),
        scratch_shapes=[pltpu.VMEM((n, h), jnp.float32)],
    )(adj, x, W1, b1.reshape(1, h), W2, dinv_col, dinv_col)

    out = pl.pallas_call(
        _agg2_body,
        grid=grid,
        in_specs=[
            pl.BlockSpec((tr, n), lambda i: (i, 0)),
            pl.BlockSpec((tr, c), lambda i: (i, 0)),
            pl.BlockSpec((1, c), lambda i: (0, 0)),
            pl.BlockSpec((n, 1), lambda i: (0, 0)),
        ],
        out_specs=pl.BlockSpec((n, c), lambda i: (0, 0)),
        out_shape=jax.ShapeDtypeStruct((n, c), jnp.float32),
        scratch_shapes=[pltpu.VMEM((n, c), jnp.float32)],
    )(adj, u2, b2.reshape(1, c), dinv_col)

    return (out, adj)


# speedup vs baseline: 50.3277x; 1.0860x over previous
import jax
import jax.numpy as jnp
from jax.experimental import pallas as pl
from jax.experimental.pallas import tpu as pltpu


def _row_tile(n):
    for t in (400, 256, 80, 64, 32, 16):
        if n % t == 0:
            return t
    return n


def _colsum_body(adj_ref, dinv_ref, adj8_ref):
    i = pl.program_id(0)

    @pl.when(i == 0)
    def _init():
        dinv_ref[...] = jnp.zeros_like(dinv_ref)

    a = adj_ref[...]
    dinv_ref[...] += jnp.sum(a, axis=0, keepdims=True)
    adj8_ref[...] = a.astype(jnp.bfloat16)

    @pl.when(i == pl.num_programs(0) - 1)
    def _fin():
        dinv_ref[...] = jax.lax.rsqrt(dinv_ref[...] + 1.0)


def _agg1_body(adj_ref, x_ref, w1_ref, b1_ref, w2_ref, dinv_ref,
               dinv_full_ref, out_ref, acc_ref):
    i = pl.program_id(0)
    tr = adj_ref.shape[0]

    @pl.when(i == 0)
    def _init():
        acc_ref[...] = jnp.zeros_like(acc_ref)

    u = jnp.dot(x_ref[...], w1_ref[...],
                preferred_element_type=jnp.float32) * dinv_ref[...]
    acc_ref[...] += jax.lax.dot_general(
        adj_ref[...].astype(jnp.bfloat16), u.astype(jnp.bfloat16),
        (((0,), (0,)), ((), ())),
        preferred_element_type=jnp.float32)
    acc_ref[pl.ds(i * tr, tr), :] += u

    @pl.when(i == pl.num_programs(0) - 1)
    def _fin():
        o1 = jnp.maximum(acc_ref[...] * dinv_full_ref[...] + b1_ref[...], 0.0)
        out_ref[...] = jnp.dot(o1, w2_ref[...],
                               preferred_element_type=jnp.float32) * dinv_full_ref[...]


def _agg2_body(adj_ref, u2_ref, b2_ref, dinv_full_ref, out_ref, acc_ref):
    i = pl.program_id(0)
    tr = adj_ref.shape[0]

    @pl.when(i == 0)
    def _init():
        acc_ref[...] = jnp.zeros_like(acc_ref)

    u = u2_ref[...]
    acc_ref[...] += jax.lax.dot_general(
        adj_ref[...].astype(jnp.bfloat16), u.astype(jnp.bfloat16),
        (((0,), (0,)), ((), ())),
        preferred_element_type=jnp.float32)
    acc_ref[pl.ds(i * tr, tr), :] += u

    @pl.when(i == pl.num_programs(0) - 1)
    def _fin():
        logits = acc_ref[...] * dinv_full_ref[...] + b2_ref[...]
        m = jnp.max(logits, axis=1, keepdims=True)
        s = logits - m
        lse = jnp.log(jnp.sum(jnp.exp(s), axis=1, keepdims=True))
        out_ref[...] = s - lse


def kernel(x, adj, W1, b1, W2, b2):
    n, d = x.shape
    h = W1.shape[1]
    c = W2.shape[1]
    tr = _row_tile(n)
    grid = (n // tr,)

    dinv, adj16 = pl.pallas_call(
        _colsum_body,
        grid=grid,
        in_specs=[pl.BlockSpec((tr, n), lambda i: (i, 0))],
        out_specs=[
            pl.BlockSpec((1, n), lambda i: (0, 0)),
            pl.BlockSpec((tr, n), lambda i: (i, 0)),
        ],
        out_shape=[
            jax.ShapeDtypeStruct((1, n), jnp.float32),
            jax.ShapeDtypeStruct((n, n), jnp.bfloat16),
        ],
    )(adj)
    dinv_col = dinv.reshape(n, 1)

    u2 = pl.pallas_call(
        _agg1_body,
        grid=grid,
        in_specs=[
            pl.BlockSpec((tr, n), lambda i: (i, 0)),
            pl.BlockSpec((tr, d), lambda i: (i, 0)),
            pl.BlockSpec((d, h), lambda i: (0, 0)),
            pl.BlockSpec((1, h), lambda i: (0, 0)),
            pl.BlockSpec((h, c), lambda i: (0, 0)),
            pl.BlockSpec((tr, 1), lambda i: (i, 0)),
            pl.BlockSpec((n, 1), lambda i: (0, 0)),
        ],
        out_specs=pl.BlockSpec((n, c), lambda i: (0, 0)),
        out_shape=jax.ShapeDtypeStruct((n, c), jnp.float32),
        scratch_shapes=[pltpu.VMEM((n, h), jnp.float32)],
    )(adj16, x, W1, b1.reshape(1, h), W2, dinv_col, dinv_col)

    out = pl.pallas_call(
        _agg2_body,
        grid=grid,
        in_specs=[
            pl.BlockSpec((tr, n), lambda i: (i, 0)),
            pl.BlockSpec((tr, c), lambda i: (i, 0)),
            pl.BlockSpec((1, c), lambda i: (0, 0)),
            pl.BlockSpec((n, 1), lambda i: (0, 0)),
        ],
        out_specs=pl.BlockSpec((n, c), lambda i: (0, 0)),
        out_shape=jax.ShapeDtypeStruct((n, c), jnp.float32),
        scratch_shapes=[pltpu.VMEM((n, c), jnp.float32)],
    )(adj16, u2, b2.reshape(1, c), dinv_col)

    return (out, adj)


# speedup vs baseline: 54.0320x; 1.0736x over previous
import jax
import jax.numpy as jnp
from jax.experimental import pallas as pl
from jax.experimental.pallas import tpu as pltpu


def _row_tile(n):
    for t in (400, 256, 80, 64, 32, 16):
        if n % t == 0:
            return t
    return n


def _colsum_body(adj_ref, dinv_ref, adj8_ref):
    i = pl.program_id(0)

    @pl.when(i == 0)
    def _init():
        dinv_ref[...] = jnp.zeros_like(dinv_ref)

    a = adj_ref[...]
    dinv_ref[...] += jnp.sum(a, axis=0, keepdims=True)
    adj8_ref[...] = a.astype(jnp.int8)

    @pl.when(i == pl.num_programs(0) - 1)
    def _fin():
        dinv_ref[...] = jax.lax.rsqrt(dinv_ref[...] + 1.0)


def _agg1_body(adj_ref, x_ref, w1_ref, b1_ref, w2_ref, dinv_ref,
               dinv_full_ref, out_ref, acc_ref):
    i = pl.program_id(0)
    tr = adj_ref.shape[0]

    @pl.when(i == 0)
    def _init():
        acc_ref[...] = jnp.zeros_like(acc_ref)

    u = jnp.dot(x_ref[...], w1_ref[...],
                preferred_element_type=jnp.float32) * dinv_ref[...]
    acc_ref[...] += jax.lax.dot_general(
        adj_ref[...].astype(jnp.bfloat16), u.astype(jnp.bfloat16),
        (((0,), (0,)), ((), ())),
        preferred_element_type=jnp.float32)
    acc_ref[pl.ds(i * tr, tr), :] += u

    @pl.when(i == pl.num_programs(0) - 1)
    def _fin():
        o1 = jnp.maximum(acc_ref[...] * dinv_full_ref[...] + b1_ref[...], 0.0)
        out_ref[...] = jnp.dot(o1, w2_ref[...],
                               preferred_element_type=jnp.float32) * dinv_full_ref[...]


def _agg2_body(adj_ref, u2_ref, b2_ref, dinv_full_ref, out_ref, acc_ref):
    i = pl.program_id(0)
    tr = adj_ref.shape[0]

    @pl.when(i == 0)
    def _init():
        acc_ref[...] = jnp.zeros_like(acc_ref)

    u = u2_ref[...]
    acc_ref[...] += jax.lax.dot_general(
        adj_ref[...].astype(jnp.bfloat16), u.astype(jnp.bfloat16),
        (((0,), (0,)), ((), ())),
        preferred_element_type=jnp.float32)
    acc_ref[pl.ds(i * tr, tr), :] += u

    @pl.when(i == pl.num_programs(0) - 1)
    def _fin():
        logits = acc_ref[...] * dinv_full_ref[...] + b2_ref[...]
        m = jnp.max(logits, axis=1, keepdims=True)
        s = logits - m
        lse = jnp.log(jnp.sum(jnp.exp(s), axis=1, keepdims=True))
        out_ref[...] = s - lse


def kernel(x, adj, W1, b1, W2, b2):
    n, d = x.shape
    h = W1.shape[1]
    c = W2.shape[1]
    tr = _row_tile(n)
    grid = (n // tr,)

    dinv, adj16 = pl.pallas_call(
        _colsum_body,
        grid=grid,
        in_specs=[pl.BlockSpec((tr, n), lambda i: (i, 0))],
        out_specs=[
            pl.BlockSpec((1, n), lambda i: (0, 0)),
            pl.BlockSpec((tr, n), lambda i: (i, 0)),
        ],
        out_shape=[
            jax.ShapeDtypeStruct((1, n), jnp.float32),
            jax.ShapeDtypeStruct((n, n), jnp.int8),
        ],
    )(adj)
    dinv_col = dinv.reshape(n, 1)

    u2 = pl.pallas_call(
        _agg1_body,
        grid=grid,
        in_specs=[
            pl.BlockSpec((tr, n), lambda i: (i, 0)),
            pl.BlockSpec((tr, d), lambda i: (i, 0)),
            pl.BlockSpec((d, h), lambda i: (0, 0)),
            pl.BlockSpec((1, h), lambda i: (0, 0)),
            pl.BlockSpec((h, c), lambda i: (0, 0)),
            pl.BlockSpec((tr, 1), lambda i: (i, 0)),
            pl.BlockSpec((n, 1), lambda i: (0, 0)),
        ],
        out_specs=pl.BlockSpec((n, c), lambda i: (0, 0)),
        out_shape=jax.ShapeDtypeStruct((n, c), jnp.float32),
        scratch_shapes=[pltpu.VMEM((n, h), jnp.float32)],
    )(adj16, x, W1, b1.reshape(1, h), W2, dinv_col, dinv_col)

    out = pl.pallas_call(
        _agg2_body,
        grid=grid,
        in_specs=[
            pl.BlockSpec((tr, n), lambda i: (i, 0)),
            pl.BlockSpec((tr, c), lambda i: (i, 0)),
            pl.BlockSpec((1, c), lambda i: (0, 0)),
            pl.BlockSpec((n, 1), lambda i: (0, 0)),
        ],
        out_specs=pl.BlockSpec((n, c), lambda i: (0, 0)),
        out_shape=jax.ShapeDtypeStruct((n, c), jnp.float32),
        scratch_shapes=[pltpu.VMEM((n, c), jnp.float32)],
    )(adj16, u2, b2.reshape(1, c), dinv_col)

    return (out, adj)


# speedup vs baseline: 55.7534x; 1.0319x over previous
import jax
import jax.numpy as jnp
from jax.experimental import pallas as pl
from jax.experimental.pallas import tpu as pltpu


def _row_tile(n):
    for t in (400, 256, 80, 64, 32, 16):
        if n % t == 0:
            return t
    return n


def _colsum_body(adj_ref, dinv_ref, adj8_ref):
    i = pl.program_id(0)

    @pl.when(i == 0)
    def _init():
        dinv_ref[...] = jnp.zeros_like(dinv_ref)

    a = adj_ref[...]
    dinv_ref[...] += jnp.sum(a, axis=0, keepdims=True)
    adj8_ref[...] = a.astype(jnp.int4)

    @pl.when(i == pl.num_programs(0) - 1)
    def _fin():
        dinv_ref[...] = jax.lax.rsqrt(dinv_ref[...] + 1.0)


def _agg1_body(adj_ref, x_ref, w1_ref, b1_ref, w2_ref, dinv_ref,
               dinv_full_ref, out_ref, acc_ref):
    i = pl.program_id(0)
    tr = adj_ref.shape[0]

    @pl.when(i == 0)
    def _init():
        acc_ref[...] = jnp.zeros_like(acc_ref)

    u = jnp.dot(x_ref[...], w1_ref[...],
                preferred_element_type=jnp.float32) * dinv_ref[...]
    acc_ref[...] += jax.lax.dot_general(
        adj_ref[...].astype(jnp.bfloat16), u.astype(jnp.bfloat16),
        (((0,), (0,)), ((), ())),
        preferred_element_type=jnp.float32)
    acc_ref[pl.ds(i * tr, tr), :] += u

    @pl.when(i == pl.num_programs(0) - 1)
    def _fin():
        o1 = jnp.maximum(acc_ref[...] * dinv_full_ref[...] + b1_ref[...], 0.0)
        out_ref[...] = jnp.dot(o1, w2_ref[...],
                               preferred_element_type=jnp.float32) * dinv_full_ref[...]


def _agg2_body(adj_ref, u2_ref, b2_ref, dinv_full_ref, out_ref, acc_ref):
    i = pl.program_id(0)
    tr = adj_ref.shape[0]

    @pl.when(i == 0)
    def _init():
        acc_ref[...] = jnp.zeros_like(acc_ref)

    u = u2_ref[...]
    acc_ref[...] += jax.lax.dot_general(
        adj_ref[...].astype(jnp.bfloat16), u.astype(jnp.bfloat16),
        (((0,), (0,)), ((), ())),
        preferred_element_type=jnp.float32)
    acc_ref[pl.ds(i * tr, tr), :] += u

    @pl.when(i == pl.num_programs(0) - 1)
    def _fin():
        logits = acc_ref[...] * dinv_full_ref[...] + b2_ref[...]
        m = jnp.max(logits, axis=1, keepdims=True)
        s = logits - m
        lse = jnp.log(jnp.sum(jnp.exp(s), axis=1, keepdims=True))
        out_ref[...] = s - lse


def kernel(x, adj, W1, b1, W2, b2):
    n, d = x.shape
    h = W1.shape[1]
    c = W2.shape[1]
    tr = _row_tile(n)
    grid = (n // tr,)

    dinv, adj16 = pl.pallas_call(
        _colsum_body,
        grid=grid,
        in_specs=[pl.BlockSpec((tr, n), lambda i: (i, 0))],
        out_specs=[
            pl.BlockSpec((1, n), lambda i: (0, 0)),
            pl.BlockSpec((tr, n), lambda i: (i, 0)),
        ],
        out_shape=[
            jax.ShapeDtypeStruct((1, n), jnp.float32),
            jax.ShapeDtypeStruct((n, n), jnp.int4),
        ],
    )(adj)
    dinv_col = dinv.reshape(n, 1)

    u2 = pl.pallas_call(
        _agg1_body,
        grid=grid,
        in_specs=[
            pl.BlockSpec((tr, n), lambda i: (i, 0)),
            pl.BlockSpec((tr, d), lambda i: (i, 0)),
            pl.BlockSpec((d, h), lambda i: (0, 0)),
            pl.BlockSpec((1, h), lambda i: (0, 0)),
            pl.BlockSpec((h, c), lambda i: (0, 0)),
            pl.BlockSpec((tr, 1), lambda i: (i, 0)),
            pl.BlockSpec((n, 1), lambda i: (0, 0)),
        ],
        out_specs=pl.BlockSpec((n, c), lambda i: (0, 0)),
        out_shape=jax.ShapeDtypeStruct((n, c), jnp.float32),
        scratch_shapes=[pltpu.VMEM((n, h), jnp.float32)],
    )(adj16, x, W1, b1.reshape(1, h), W2, dinv_col, dinv_col)

    out = pl.pallas_call(
        _agg2_body,
        grid=grid,
        in_specs=[
            pl.BlockSpec((tr, n), lambda i: (i, 0)),
            pl.BlockSpec((tr, c), lambda i: (i, 0)),
            pl.BlockSpec((1, c), lambda i: (0, 0)),
            pl.BlockSpec((n, 1), lambda i: (0, 0)),
        ],
        out_specs=pl.BlockSpec((n, c), lambda i: (0, 0)),
        out_shape=jax.ShapeDtypeStruct((n, c), jnp.float32),
        scratch_shapes=[pltpu.VMEM((n, c), jnp.float32)],
    )(adj16, u2, b2.reshape(1, c), dinv_col)

    return (out, adj)


# speedup vs baseline: 62.3702x; 1.1187x over previous
import jax
import jax.numpy as jnp
from jax.experimental import pallas as pl
from jax.experimental.pallas import tpu as pltpu


def _row_tile(n):
    for t in (400, 256, 80, 64, 32, 16):
        if n % t == 0:
            return t
    return n


def _pass1_body(adj_ref, dinv_ref, adj4_ref):
    i = pl.program_id(0)

    @pl.when(i == 0)
    def _init():
        dinv_ref[...] = jnp.zeros_like(dinv_ref)

    a = adj_ref[...]
    dinv_ref[...] += jnp.sum(a, axis=0, keepdims=True)
    adj4_ref[...] = a.astype(jnp.int4)

    @pl.when(i == pl.num_programs(0) - 1)
    def _fin():
        dinv_ref[...] = jax.lax.rsqrt(dinv_ref[...] + 1.0)


def _uprep_body(xt_ref, w1t_ref, dinv_ref, u1t_ref):
    u1t_ref[0] = jnp.dot(w1t_ref[...], xt_ref[0],
                         preferred_element_type=jnp.float32) * dinv_ref[0]


def _agg1_body(adj_ref, u1t_tile_ref, u1t_full_ref, b1_ref, w2t_ref,
               dinv_ref, out_ref, acc_ref):
    i = pl.program_id(0)

    @pl.when(i == 0)
    def _init():
        acc_ref[...] = jnp.zeros_like(acc_ref)

    acc_ref[...] += jax.lax.dot_general(
        u1t_tile_ref[0].astype(jnp.bfloat16),
        adj_ref[...].astype(jnp.bfloat16),
        (((1,), (0,)), ((), ())),
        preferred_element_type=jnp.float32)

    @pl.when(i == pl.num_programs(0) - 1)
    def _fin():
        acc = acc_ref[...] + u1t_full_ref[...]
        o1 = jnp.maximum(acc * dinv_ref[...] + b1_ref[...], 0.0)
        out_ref[...] = jnp.dot(w2t_ref[...], o1,
                               preferred_element_type=jnp.float32) * dinv_ref[...]


def _agg2_body(adj_ref, u2t_tile_ref, u2t_full_ref, b2_ref, dinv_ref,
               out_ref, acc_ref):
    i = pl.program_id(0)

    @pl.when(i == 0)
    def _init():
        acc_ref[...] = jnp.zeros_like(acc_ref)

    acc_ref[...] += jax.lax.dot_general(
        u2t_tile_ref[0].astype(jnp.bfloat16),
        adj_ref[...].astype(jnp.bfloat16),
        (((1,), (0,)), ((), ())),
        preferred_element_type=jnp.float32)

    @pl.when(i == pl.num_programs(0) - 1)
    def _fin():
        logits = (acc_ref[...] + u2t_full_ref[...]) * dinv_ref[...] + b2_ref[...]
        m = jnp.max(logits, axis=0, keepdims=True)
        s = logits - m
        lse = jnp.log(jnp.sum(jnp.exp(s), axis=0, keepdims=True))
        out_ref[...] = s - lse


def kernel(x, adj, W1, b1, W2, b2):
    n, d = x.shape
    h = W1.shape[1]
    c = W2.shape[1]
    tr = _row_tile(n)
    nt = n // tr
    grid = (nt,)

    dinv, adj4 = pl.pallas_call(
        _pass1_body,
        grid=grid,
        in_specs=[pl.BlockSpec((tr, n), lambda i: (i, 0))],
        out_specs=[
            pl.BlockSpec((1, n), lambda i: (0, 0)),
            pl.BlockSpec((tr, n), lambda i: (i, 0)),
        ],
        out_shape=[
            jax.ShapeDtypeStruct((1, n), jnp.float32),
            jax.ShapeDtypeStruct((n, n), jnp.int4),
        ],
    )(adj)

    xt3 = x.reshape(nt, tr, d).transpose(0, 2, 1)
    dinv3 = dinv.reshape(nt, 1, tr)

    u1t3 = pl.pallas_call(
        _uprep_body,
        grid=grid,
        in_specs=[
            pl.BlockSpec((1, d, tr), lambda i: (i, 0, 0)),
            pl.BlockSpec((h, d), lambda i: (0, 0)),
            pl.BlockSpec((1, 1, tr), lambda i: (i, 0, 0)),
        ],
        out_specs=pl.BlockSpec((1, h, tr), lambda i: (i, 0, 0)),
        out_shape=jax.ShapeDtypeStruct((nt, h, tr), jnp.float32),
    )(xt3, W1.T, dinv3)
    u1t = u1t3.transpose(1, 0, 2).reshape(h, n)

    u2t = pl.pallas_call(
        _agg1_body,
        grid=grid,
        in_specs=[
            pl.BlockSpec((tr, n), lambda i: (i, 0)),
            pl.BlockSpec((1, h, tr), lambda i: (i, 0, 0)),
            pl.BlockSpec((h, n), lambda i: (0, 0)),
            pl.BlockSpec((h, 1), lambda i: (0, 0)),
            pl.BlockSpec((c, h), lambda i: (0, 0)),
            pl.BlockSpec((1, n), lambda i: (0, 0)),
        ],
        out_specs=pl.BlockSpec((c, n), lambda i: (0, 0)),
        out_shape=jax.ShapeDtypeStruct((c, n), jnp.float32),
        scratch_shapes=[pltpu.VMEM((h, n), jnp.float32)],
    )(adj4, u1t3, u1t, b1.reshape(h, 1), W2.T, dinv)

    u2t3 = u2t.reshape(c, nt, tr).transpose(1, 0, 2)

    outt = pl.pallas_call(
        _agg2_body,
        grid=grid,
        in_specs=[
            pl.BlockSpec((tr, n), lambda i: (i, 0)),
            pl.BlockSpec((1, c, tr), lambda i: (i, 0, 0)),
            pl.BlockSpec((c, n), lambda i: (0, 0)),
            pl.BlockSpec((c, 1), lambda i: (0, 0)),
            pl.BlockSpec((1, n), lambda i: (0, 0)),
        ],
        out_specs=pl.BlockSpec((c, n), lambda i: (0, 0)),
        out_shape=jax.ShapeDtypeStruct((c, n), jnp.float32),
        scratch_shapes=[pltpu.VMEM((c, n), jnp.float32)],
    )(adj4, u2t3, u2t, b2.reshape(c, 1), dinv)

    return (outt.T, adj)
